# baseline (speedup 1.0000x reference)
import functools

import jax
import jax.numpy as jnp
from jax import lax
from jax.experimental import pallas as pl
from jax.experimental.pallas import tpu as pltpu
from jax.experimental.pallas import tpu_sc as plsc

_GROUPS = 8
_NUM_VARS = 64
_VAR_DIM = 64
_PROJ_DIM = 32
_PAIRS = 4
_PAIR_DIM = 2 * _VAR_DIM

_BLK = 768

_info = plsc.get_sparse_core_info()
_NC = _info.num_cores
_NS = _info.num_subcores
_NW = _NC * _NS
_CHUNK = 128


def _pair_table_body(cb_ref, pt_ref):
    cb0 = cb_ref[:_NUM_VARS, :]
    cb1 = cb_ref[_NUM_VARS:, :]
    a = jnp.broadcast_to(cb0[:, None, :], (_NUM_VARS, _NUM_VARS, _VAR_DIM))
    b = jnp.broadcast_to(cb1[None, :, :], (_NUM_VARS, _NUM_VARS, _VAR_DIM))
    a = a.reshape(_NUM_VARS * _NUM_VARS, _VAR_DIM)
    b = b.reshape(_NUM_VARS * _NUM_VARS, _VAR_DIM)
    pt_ref[...] = jnp.concatenate([a, b], axis=-1)


def _logits_argmax_body(x_ref, wpre_ref, bpre_ref, wwp_ref, bwp_ref, pow2_ref,
                        idx_ref):
    x = x_ref[...]
    h = jnp.dot(x, wpre_ref[...], preferred_element_type=jnp.float32)
    h = h + bpre_ref[...]
    logits = jnp.dot(h, wwp_ref[...], preferred_element_type=jnp.float32)
    logits = logits + bwp_ref[...]
    rows = x.shape[0]
    mfull = jnp.concatenate(
        [jnp.broadcast_to(
            jnp.max(logits[:, g * _NUM_VARS:(g + 1) * _NUM_VARS], axis=-1,
                    keepdims=True), (rows, _NUM_VARS))
         for g in range(_GROUPS)], axis=-1)
    eqb = (logits == mfull).astype(jnp.bfloat16)
    y = jnp.dot(eqb, pow2_ref[...], preferred_element_type=jnp.float32)
    e = jax.lax.bitcast_convert_type(y, jnp.int32) >> 23
    k_all = jnp.clip(jnp.int32(63 + 127) - e, 0, _NUM_VARS - 1)
    parts = []
    for p in range(_PAIRS):
        pidx = (k_all[:, 2 * p] * _NUM_VARS + k_all[:, 2 * p + 1]
                + p * _NUM_VARS * _NUM_VARS)
        parts.append(pidx.reshape(rows // _CHUNK, _CHUNK))
    idx_ref[...] = jnp.concatenate(parts, axis=0)


def _postproject_body(q_ref, wp_ref, bpost_ref, out_ref):
    r = q_ref[...].reshape(_PAIRS, _BLK, _PAIR_DIM)
    acc = None
    for p in range(_PAIRS):
        qp = r[p].astype(jnp.bfloat16)
        wp = wp_ref[p * _PAIR_DIM:(p + 1) * _PAIR_DIM, :].astype(jnp.bfloat16)
        d = jnp.dot(qp, wp, preferred_element_type=jnp.float32)
        acc = d if acc is None else acc + d
    out_ref[...] = acc + bpost_ref[...]


def _postproject_alias_body(q_ref, wp_ref, bpost_ref, prev_ref, out_ref):
    del prev_ref
    _postproject_body(q_ref, wp_ref, bpost_ref, out_ref)


def _make_sc_gather(rows, nch):
    mesh = plsc.VectorSubcoreMesh(core_axis_name="c", subcore_axis_name="s")
    nrows_idx = rows // _CHUNK

    @functools.partial(
        pl.kernel,
        mesh=mesh,
        out_type=jax.ShapeDtypeStruct((rows, _PAIR_DIM), jnp.float32),
        scratch_types=[
            pltpu.VMEM((nrows_idx, _CHUNK), jnp.int32),
            pltpu.VMEM((_CHUNK, _PAIR_DIM), jnp.float32),
            pltpu.VMEM((_CHUNK, _PAIR_DIM), jnp.float32),
            pltpu.SemaphoreType.DMA,
            pltpu.SemaphoreType.DMA,
        ],
    )
    def gather(idx_hbm, pt_hbm, out_hbm, idx_v, buf0, buf1, sem0, sem1):
        wid = lax.axis_index("s") * _NC + lax.axis_index("c")
        pltpu.sync_copy(idx_hbm, idx_v)
        bufs = (buf0, buf1)
        sems = (sem0, sem1)
        base = wid * nch * _CHUNK
        row0 = wid * nch
        cps = [None] * nch
        cps[0] = pltpu.async_copy(pt_hbm.at[idx_v.at[row0]], bufs[0], sems[0])
        for j in range(nch):
            if j + 1 < nch:
                cps[j + 1] = pltpu.async_copy(
                    pt_hbm.at[idx_v.at[row0 + j + 1]],
                    bufs[(j + 1) % 2], sems[(j + 1) % 2])
            cps[j].wait()
            pltpu.sync_copy(bufs[j % 2],
                            out_hbm.at[pl.ds(base + j * _CHUNK, _CHUNK)])

    return gather


def kernel(x, W_pre, b_pre, W_wp, b_wp, codebook, W_post, b_post):
    B, T, IN_DIM = x.shape
    OUT_DIM = W_post.shape[1]
    BT = B * T
    xf = x.reshape(BT, IN_DIM)
    bpre2 = b_pre.reshape(1, -1)
    bwp2 = b_wp.reshape(1, -1)
    bpost2 = b_post.reshape(1, -1)
    NN = _NUM_VARS * _NUM_VARS

    pair_table = pl.pallas_call(
        _pair_table_body,
        grid=(_PAIRS,),
        in_specs=[pl.BlockSpec((2 * _NUM_VARS, _VAR_DIM), lambda i: (i, 0))],
        out_specs=pl.BlockSpec((NN, _PAIR_DIM), lambda i: (i, 0)),
        out_shape=jax.ShapeDtypeStruct((_PAIRS * NN, _PAIR_DIM), jnp.float32),
    )(codebook)

    THIRDS = 3
    BT3 = BT // THIRDS
    G3 = BT3 // _BLK
    rows3 = BT3 * _PAIRS
    nch = rows3 // (_NW * _CHUNK)
    sc_gather = _make_sc_gather(rows3, nch)

    vv = jnp.arange(_GROUPS * _NUM_VARS)
    pw = jnp.exp2((_NUM_VARS - 1 - (vv % _NUM_VARS)).astype(jnp.float32))
    pow2 = jnp.where((vv // _NUM_VARS)[:, None] == jnp.arange(_GROUPS)[None, :],
                     pw[:, None], 0.0).astype(jnp.bfloat16)

    def argmax_call(h):
        return pl.pallas_call(
            _logits_argmax_body,
            grid=(G3,),
            in_specs=[
                pl.BlockSpec((_BLK, IN_DIM), lambda i, h=h: (i + G3 * h, 0)),
                pl.BlockSpec((IN_DIM, _PROJ_DIM), lambda i: (0, 0)),
                pl.BlockSpec((1, _PROJ_DIM), lambda i: (0, 0)),
                pl.BlockSpec((_PROJ_DIM, _GROUPS * _NUM_VARS), lambda i: (0, 0)),
                pl.BlockSpec((1, _GROUPS * _NUM_VARS), lambda i: (0, 0)),
                pl.BlockSpec((_GROUPS * _NUM_VARS, _GROUPS), lambda i: (0, 0)),
            ],
            out_specs=pl.BlockSpec((_BLK * _PAIRS // _CHUNK, _CHUNK),
                                   lambda i: (i, 0)),
            out_shape=jax.ShapeDtypeStruct((rows3 // _CHUNK, _CHUNK), jnp.int32),
        )(xf, W_pre, bpre2, W_wp, bwp2, pow2)

    def post_call(h, q, prev):
        qspec = pl.BlockSpec((_BLK * _PAIRS, _PAIR_DIM), lambda i: (i, 0))
        wspec = pl.BlockSpec((_PAIRS * _PAIR_DIM, OUT_DIM), lambda i: (0, 0))
        bspec = pl.BlockSpec((1, OUT_DIM), lambda i: (0, 0))
        ospec = pl.BlockSpec((_BLK, OUT_DIM), lambda i, h=h: (i + G3 * h, 0))
        oshape = jax.ShapeDtypeStruct((BT, OUT_DIM), jnp.float32)
        if prev is None:
            return pl.pallas_call(
                _postproject_body, grid=(G3,),
                in_specs=[qspec, wspec, bspec],
                out_specs=ospec, out_shape=oshape,
            )(q, W_post, bpost2)
        return pl.pallas_call(
            _postproject_alias_body, grid=(G3,),
            in_specs=[qspec, wspec, bspec,
                      pl.BlockSpec(memory_space=pl.ANY)],
            out_specs=ospec, out_shape=oshape,
            input_output_aliases={3: 0},
        )(q, W_post, bpost2, prev)

    idxs = [argmax_call(h) for h in range(THIRDS)]
    qs = [sc_gather(idxs[h], pair_table)
          for h in range(THIRDS)]
    out = None
    for h in range(THIRDS):
        out = post_call(h, qs[h], out)

    return out.reshape(B, T, OUT_DIM)


# speedup vs baseline: 1.0040x; 1.0040x over previous
import functools

import jax
import jax.numpy as jnp
from jax import lax
from jax.experimental import pallas as pl
from jax.experimental.pallas import tpu as pltpu
from jax.experimental.pallas import tpu_sc as plsc

_GROUPS = 8
_NUM_VARS = 64
_VAR_DIM = 64
_PROJ_DIM = 32
_PAIRS = 4
_PAIR_DIM = 2 * _VAR_DIM

_BLK = 1024

_info = plsc.get_sparse_core_info()
_NC = _info.num_cores
_NS = _info.num_subcores
_NW = _NC * _NS
_CHUNK = 128


def _pair_table_body(cb_ref, pt_ref):
    cb0 = cb_ref[:_NUM_VARS, :]
    cb1 = cb_ref[_NUM_VARS:, :]
    a = jnp.broadcast_to(cb0[:, None, :], (_NUM_VARS, _NUM_VARS, _VAR_DIM))
    b = jnp.broadcast_to(cb1[None, :, :], (_NUM_VARS, _NUM_VARS, _VAR_DIM))
    a = a.reshape(_NUM_VARS * _NUM_VARS, _VAR_DIM)
    b = b.reshape(_NUM_VARS * _NUM_VARS, _VAR_DIM)
    pt_ref[...] = jnp.concatenate([a, b], axis=-1)


def _logits_argmax_body(x_ref, wpre_ref, bpre_ref, wwp_ref, bwp_ref, pow2_ref,
                        idx_ref):
    x = x_ref[...]
    h = jnp.dot(x, wpre_ref[...], preferred_element_type=jnp.float32)
    h = h + bpre_ref[...]
    logits = jnp.dot(h, wwp_ref[...], preferred_element_type=jnp.float32)
    logits = logits + bwp_ref[...]
    rows = x.shape[0]
    mfull = jnp.concatenate(
        [jnp.broadcast_to(
            jnp.max(logits[:, g * _NUM_VARS:(g + 1) * _NUM_VARS], axis=-1,
                    keepdims=True), (rows, _NUM_VARS))
         for g in range(_GROUPS)], axis=-1)
    eqb = (logits == mfull).astype(jnp.bfloat16)
    y = jnp.dot(eqb, pow2_ref[...], preferred_element_type=jnp.float32)
    e = jax.lax.bitcast_convert_type(y, jnp.int32) >> 23
    k_all = jnp.clip(jnp.int32(63 + 127) - e, 0, _NUM_VARS - 1)
    parts = []
    for p in range(_PAIRS):
        pidx = (k_all[:, 2 * p] * _NUM_VARS + k_all[:, 2 * p + 1]
                + p * _NUM_VARS * _NUM_VARS)
        parts.append(pidx.reshape(rows // _CHUNK, _CHUNK))
    idx_ref[...] = jnp.concatenate(parts, axis=0)


def _postproject_body(q_ref, wp_ref, bpost_ref, out_ref):
    r = q_ref[...].reshape(_PAIRS, _BLK, _PAIR_DIM)
    acc = None
    for p in range(_PAIRS):
        qp = r[p].astype(jnp.bfloat16)
        wp = wp_ref[p * _PAIR_DIM:(p + 1) * _PAIR_DIM, :].astype(jnp.bfloat16)
        d = jnp.dot(qp, wp, preferred_element_type=jnp.float32)
        acc = d if acc is None else acc + d
    out_ref[...] = acc + bpost_ref[...]


def _postproject_alias_body(q_ref, wp_ref, bpost_ref, prev_ref, out_ref):
    del prev_ref
    _postproject_body(q_ref, wp_ref, bpost_ref, out_ref)


def _make_sc_gather(rows, nch):
    mesh = plsc.VectorSubcoreMesh(core_axis_name="c", subcore_axis_name="s")
    nrows_idx = rows // _CHUNK

    @functools.partial(
        pl.kernel,
        mesh=mesh,
        out_type=jax.ShapeDtypeStruct((rows, _PAIR_DIM), jnp.float32),
        scratch_types=[
            pltpu.VMEM((nrows_idx, _CHUNK), jnp.int32),
            pltpu.VMEM((_CHUNK, _PAIR_DIM), jnp.float32),
            pltpu.VMEM((_CHUNK, _PAIR_DIM), jnp.float32),
            pltpu.SemaphoreType.DMA,
            pltpu.SemaphoreType.DMA,
        ],
    )
    def gather(idx_hbm, pt_hbm, out_hbm, idx_v, buf0, buf1, sem0, sem1):
        wid = lax.axis_index("s") * _NC + lax.axis_index("c")
        pltpu.sync_copy(idx_hbm, idx_v)
        bufs = (buf0, buf1)
        sems = (sem0, sem1)
        base = wid * nch * _CHUNK
        row0 = wid * nch
        cps = [None] * nch
        cps[0] = pltpu.async_copy(pt_hbm.at[idx_v.at[row0]], bufs[0], sems[0])
        for j in range(nch):
            if j + 1 < nch:
                cps[j + 1] = pltpu.async_copy(
                    pt_hbm.at[idx_v.at[row0 + j + 1]],
                    bufs[(j + 1) % 2], sems[(j + 1) % 2])
            cps[j].wait()
            pltpu.sync_copy(bufs[j % 2],
                            out_hbm.at[pl.ds(base + j * _CHUNK, _CHUNK)])

    return gather


def kernel(x, W_pre, b_pre, W_wp, b_wp, codebook, W_post, b_post):
    B, T, IN_DIM = x.shape
    OUT_DIM = W_post.shape[1]
    BT = B * T
    xf = x.reshape(BT, IN_DIM)
    bpre2 = b_pre.reshape(1, -1)
    bwp2 = b_wp.reshape(1, -1)
    bpost2 = b_post.reshape(1, -1)
    NN = _NUM_VARS * _NUM_VARS

    pair_table = pl.pallas_call(
        _pair_table_body,
        grid=(_PAIRS,),
        in_specs=[pl.BlockSpec((2 * _NUM_VARS, _VAR_DIM), lambda i: (i, 0))],
        out_specs=pl.BlockSpec((NN, _PAIR_DIM), lambda i: (i, 0)),
        out_shape=jax.ShapeDtypeStruct((_PAIRS * NN, _PAIR_DIM), jnp.float32),
    )(codebook)

    THIRDS = 3
    BT3 = BT // THIRDS
    G3 = BT3 // _BLK
    rows3 = BT3 * _PAIRS
    nch = rows3 // (_NW * _CHUNK)
    sc_gather = _make_sc_gather(rows3, nch)

    vv = jnp.arange(_GROUPS * _NUM_VARS)
    pw = jnp.exp2((_NUM_VARS - 1 - (vv % _NUM_VARS)).astype(jnp.float32))
    pow2 = jnp.where((vv // _NUM_VARS)[:, None] == jnp.arange(_GROUPS)[None, :],
                     pw[:, None], 0.0).astype(jnp.bfloat16)

    def argmax_call(h):
        return pl.pallas_call(
            _logits_argmax_body,
            grid=(G3,),
            in_specs=[
                pl.BlockSpec((_BLK, IN_DIM), lambda i, h=h: (i + G3 * h, 0)),
                pl.BlockSpec((IN_DIM, _PROJ_DIM), lambda i: (0, 0)),
                pl.BlockSpec((1, _PROJ_DIM), lambda i: (0, 0)),
                pl.BlockSpec((_PROJ_DIM, _GROUPS * _NUM_VARS), lambda i: (0, 0)),
                pl.BlockSpec((1, _GROUPS * _NUM_VARS), lambda i: (0, 0)),
                pl.BlockSpec((_GROUPS * _NUM_VARS, _GROUPS), lambda i: (0, 0)),
            ],
            out_specs=pl.BlockSpec((_BLK * _PAIRS // _CHUNK, _CHUNK),
                                   lambda i: (i, 0)),
            out_shape=jax.ShapeDtypeStruct((rows3 // _CHUNK, _CHUNK), jnp.int32),
        )(xf, W_pre, bpre2, W_wp, bwp2, pow2)

    def post_call(h, q, prev):
        qspec = pl.BlockSpec((_BLK * _PAIRS, _PAIR_DIM), lambda i: (i, 0))
        wspec = pl.BlockSpec((_PAIRS * _PAIR_DIM, OUT_DIM), lambda i: (0, 0))
        bspec = pl.BlockSpec((1, OUT_DIM), lambda i: (0, 0))
        ospec = pl.BlockSpec((_BLK, OUT_DIM), lambda i, h=h: (i + G3 * h, 0))
        oshape = jax.ShapeDtypeStruct((BT, OUT_DIM), jnp.float32)
        if prev is None:
            return pl.pallas_call(
                _postproject_body, grid=(G3,),
                in_specs=[qspec, wspec, bspec],
                out_specs=ospec, out_shape=oshape,
            )(q, W_post, bpost2)
        return pl.pallas_call(
            _postproject_alias_body, grid=(G3,),
            in_specs=[qspec, wspec, bspec,
                      pl.BlockSpec(memory_space=pl.ANY)],
            out_specs=ospec, out_shape=oshape,
            input_output_aliases={3: 0},
        )(q, W_post, bpost2, prev)

    idxs = [argmax_call(h) for h in range(THIRDS)]
    qs = [sc_gather(idxs[h], pair_table)
          for h in range(THIRDS)]
    out = None
    for h in range(THIRDS):
        out = post_call(h, qs[h], out)

    return out.reshape(B, T, OUT_DIM)


# speedup vs baseline: 1.0060x; 1.0020x over previous
import functools

import numpy as np

import jax
import jax.numpy as jnp
from jax import lax
from jax.experimental import pallas as pl
from jax.experimental.pallas import tpu as pltpu
from jax.experimental.pallas import tpu_sc as plsc

_GROUPS = 8
_NUM_VARS = 64
_VAR_DIM = 64
_PROJ_DIM = 32
_PAIRS = 4
_PAIR_DIM = 2 * _VAR_DIM

_BLK = 1024

_info = plsc.get_sparse_core_info()
_NC = _info.num_cores
_NS = _info.num_subcores
_NW = _NC * _NS
_CHUNK = 128

_VV = np.arange(_GROUPS * _NUM_VARS)
_POW2 = np.where((_VV // _NUM_VARS)[:, None] == np.arange(_GROUPS)[None, :],
                 np.exp2(_NUM_VARS - 1 - (_VV % _NUM_VARS))[:, None],
                 0.0).astype(np.float32)


def _pair_table_body(cb_ref, pt_ref):
    cb0 = cb_ref[:_NUM_VARS, :]
    cb1 = cb_ref[_NUM_VARS:, :]
    a = jnp.broadcast_to(cb0[:, None, :], (_NUM_VARS, _NUM_VARS, _VAR_DIM))
    b = jnp.broadcast_to(cb1[None, :, :], (_NUM_VARS, _NUM_VARS, _VAR_DIM))
    a = a.reshape(_NUM_VARS * _NUM_VARS, _VAR_DIM)
    b = b.reshape(_NUM_VARS * _NUM_VARS, _VAR_DIM)
    pt_ref[...] = jnp.concatenate([a, b], axis=-1)


def _logits_argmax_body(x_ref, wpre_ref, bpre_ref, wwp_ref, bwp_ref, pow2_ref,
                        idx_ref):
    x = x_ref[...]
    h = jnp.dot(x, wpre_ref[...], preferred_element_type=jnp.float32)
    h = h + bpre_ref[...]
    logits = jnp.dot(h, wwp_ref[...], preferred_element_type=jnp.float32)
    logits = logits + bwp_ref[...]
    rows = x.shape[0]
    mfull = jnp.concatenate(
        [jnp.broadcast_to(
            jnp.max(logits[:, g * _NUM_VARS:(g + 1) * _NUM_VARS], axis=-1,
                    keepdims=True), (rows, _NUM_VARS))
         for g in range(_GROUPS)], axis=-1)
    eqb = (logits == mfull).astype(jnp.bfloat16)
    y = jnp.dot(eqb, pow2_ref[...], preferred_element_type=jnp.float32)
    e = jax.lax.bitcast_convert_type(y, jnp.int32) >> 23
    k_all = jnp.clip(jnp.int32(63 + 127) - e, 0, _NUM_VARS - 1)
    parts = []
    for p in range(_PAIRS):
        pidx = (k_all[:, 2 * p] * _NUM_VARS + k_all[:, 2 * p + 1]
                + p * _NUM_VARS * _NUM_VARS)
        parts.append(pidx.reshape(rows // _CHUNK, _CHUNK))
    idx_ref[...] = jnp.concatenate(parts, axis=0)


def _postproject_body(q_ref, wp_ref, bpost_ref, out_ref):
    r = q_ref[...].reshape(_PAIRS, _BLK, _PAIR_DIM)
    acc = None
    for p in range(_PAIRS):
        qp = r[p].astype(jnp.bfloat16)
        wp = wp_ref[p * _PAIR_DIM:(p + 1) * _PAIR_DIM, :].astype(jnp.bfloat16)
        d = jnp.dot(qp, wp, preferred_element_type=jnp.float32)
        acc = d if acc is None else acc + d
    out_ref[...] = acc + bpost_ref[...]


def _postproject_alias_body(q_ref, wp_ref, bpost_ref, prev_ref, out_ref):
    del prev_ref
    _postproject_body(q_ref, wp_ref, bpost_ref, out_ref)


def _make_sc_gather(rows, nch):
    mesh = plsc.VectorSubcoreMesh(core_axis_name="c", subcore_axis_name="s")
    nrows_idx = rows // _CHUNK

    @functools.partial(
        pl.kernel,
        mesh=mesh,
        out_type=jax.ShapeDtypeStruct((rows, _PAIR_DIM), jnp.float32),
        scratch_types=[
            pltpu.VMEM((nrows_idx, _CHUNK), jnp.int32),
            pltpu.VMEM((_CHUNK, _PAIR_DIM), jnp.float32),
            pltpu.VMEM((_CHUNK, _PAIR_DIM), jnp.float32),
            pltpu.SemaphoreType.DMA,
            pltpu.SemaphoreType.DMA,
        ],
    )
    def gather(idx_hbm, pt_hbm, out_hbm, idx_v, buf0, buf1, sem0, sem1):
        wid = lax.axis_index("s") * _NC + lax.axis_index("c")
        pltpu.sync_copy(idx_hbm, idx_v)
        bufs = (buf0, buf1)
        sems = (sem0, sem1)
        base = wid * nch * _CHUNK
        row0 = wid * nch
        cps = [None] * nch
        cps[0] = pltpu.async_copy(pt_hbm.at[idx_v.at[row0]], bufs[0], sems[0])
        for j in range(nch):
            if j + 1 < nch:
                cps[j + 1] = pltpu.async_copy(
                    pt_hbm.at[idx_v.at[row0 + j + 1]],
                    bufs[(j + 1) % 2], sems[(j + 1) % 2])
            cps[j].wait()
            pltpu.sync_copy(bufs[j % 2],
                            out_hbm.at[pl.ds(base + j * _CHUNK, _CHUNK)])

    return gather


def kernel(x, W_pre, b_pre, W_wp, b_wp, codebook, W_post, b_post):
    B, T, IN_DIM = x.shape
    OUT_DIM = W_post.shape[1]
    BT = B * T
    xf = x.reshape(BT, IN_DIM)
    bpre2 = b_pre.reshape(1, -1)
    bwp2 = b_wp.reshape(1, -1)
    bpost2 = b_post.reshape(1, -1)
    NN = _NUM_VARS * _NUM_VARS

    pair_table = pl.pallas_call(
        _pair_table_body,
        grid=(_PAIRS,),
        in_specs=[pl.BlockSpec((2 * _NUM_VARS, _VAR_DIM), lambda i: (i, 0))],
        out_specs=pl.BlockSpec((NN, _PAIR_DIM), lambda i: (i, 0)),
        out_shape=jax.ShapeDtypeStruct((_PAIRS * NN, _PAIR_DIM), jnp.float32),
    )(codebook)

    THIRDS = 3
    BT3 = BT // THIRDS
    G3 = BT3 // _BLK
    rows3 = BT3 * _PAIRS
    nch = rows3 // (_NW * _CHUNK)
    sc_gather = _make_sc_gather(rows3, nch)

    pow2 = jnp.asarray(_POW2).astype(jnp.bfloat16)

    def argmax_call(h):
        return pl.pallas_call(
            _logits_argmax_body,
            grid=(G3,),
            in_specs=[
                pl.BlockSpec((_BLK, IN_DIM), lambda i, h=h: (i + G3 * h, 0)),
                pl.BlockSpec((IN_DIM, _PROJ_DIM), lambda i: (0, 0)),
                pl.BlockSpec((1, _PROJ_DIM), lambda i: (0, 0)),
                pl.BlockSpec((_PROJ_DIM, _GROUPS * _NUM_VARS), lambda i: (0, 0)),
                pl.BlockSpec((1, _GROUPS * _NUM_VARS), lambda i: (0, 0)),
                pl.BlockSpec((_GROUPS * _NUM_VARS, _GROUPS), lambda i: (0, 0)),
            ],
            out_specs=pl.BlockSpec((_BLK * _PAIRS // _CHUNK, _CHUNK),
                                   lambda i: (i, 0)),
            out_shape=jax.ShapeDtypeStruct((rows3 // _CHUNK, _CHUNK), jnp.int32),
        )(xf, W_pre, bpre2, W_wp, bwp2, pow2)

    def post_call(h, q, prev):
        qspec = pl.BlockSpec((_BLK * _PAIRS, _PAIR_DIM), lambda i: (i, 0))
        wspec = pl.BlockSpec((_PAIRS * _PAIR_DIM, OUT_DIM), lambda i: (0, 0))
        bspec = pl.BlockSpec((1, OUT_DIM), lambda i: (0, 0))
        ospec = pl.BlockSpec((_BLK, OUT_DIM), lambda i, h=h: (i + G3 * h, 0))
        oshape = jax.ShapeDtypeStruct((BT, OUT_DIM), jnp.float32)
        if prev is None:
            return pl.pallas_call(
                _postproject_body, grid=(G3,),
                in_specs=[qspec, wspec, bspec],
                out_specs=ospec, out_shape=oshape,
            )(q, W_post, bpost2)
        return pl.pallas_call(
            _postproject_alias_body, grid=(G3,),
            in_specs=[qspec, wspec, bspec,
                      pl.BlockSpec(memory_space=pl.ANY)],
            out_specs=ospec, out_shape=oshape,
            input_output_aliases={3: 0},
        )(q, W_post, bpost2, prev)

    idxs = [argmax_call(h) for h in range(THIRDS)]
    qs = [sc_gather(idxs[h], pair_table)
          for h in range(THIRDS)]
    out = None
    for h in range(THIRDS):
        out = post_call(h, qs[h], out)

    return out.reshape(B, T, OUT_DIM)


# speedup vs baseline: 1.0495x; 1.0432x over previous
import functools

import numpy as np

import jax
import jax.numpy as jnp
from jax import lax
from jax.experimental import pallas as pl
from jax.experimental.pallas import tpu as pltpu
from jax.experimental.pallas import tpu_sc as plsc

_GROUPS = 8
_NUM_VARS = 64
_VAR_DIM = 64
_PROJ_DIM = 32
_PAIRS = 4
_PAIR_DIM = 2 * _VAR_DIM

_BLK = 1024

_info = plsc.get_sparse_core_info()
_NC = _info.num_cores
_NS = _info.num_subcores
_NW = _NC * _NS
_CHUNK = 128

_VV = np.arange(_GROUPS * _NUM_VARS)
_POW2 = np.where((_VV // _NUM_VARS)[:, None] == np.arange(_GROUPS)[None, :],
                 np.exp2(_NUM_VARS - 1 - (_VV % _NUM_VARS))[:, None],
                 0.0).astype(np.float32)


def _pair_table_body(cb_ref, pt_ref):
    cb0 = cb_ref[:_NUM_VARS, :]
    cb1 = cb_ref[_NUM_VARS:, :]
    a = jnp.broadcast_to(cb0[:, None, :], (_NUM_VARS, _NUM_VARS, _VAR_DIM))
    b = jnp.broadcast_to(cb1[None, :, :], (_NUM_VARS, _NUM_VARS, _VAR_DIM))
    a = a.reshape(_NUM_VARS * _NUM_VARS, _VAR_DIM)
    b = b.reshape(_NUM_VARS * _NUM_VARS, _VAR_DIM)
    pt_ref[...] = jnp.concatenate([a, b], axis=-1)


def _logits_argmax_body(x_ref, wpre_ref, bpre_ref, wwp_ref, bwp_ref, pow2_ref,
                        idx_ref):
    x = x_ref[...]
    h = jnp.dot(x, wpre_ref[...], preferred_element_type=jnp.float32)
    h = h + bpre_ref[...]
    logits = jnp.dot(h, wwp_ref[...], preferred_element_type=jnp.float32)
    logits = logits + bwp_ref[...]
    rows = x.shape[0]
    mfull = jnp.concatenate(
        [jnp.broadcast_to(
            jnp.max(logits[:, g * _NUM_VARS:(g + 1) * _NUM_VARS], axis=-1,
                    keepdims=True), (rows, _NUM_VARS))
         for g in range(_GROUPS)], axis=-1)
    eqb = (logits == mfull).astype(jnp.bfloat16)
    y = jnp.dot(eqb, pow2_ref[...], preferred_element_type=jnp.float32)
    e = jax.lax.bitcast_convert_type(y, jnp.int32) >> 23
    k_all = jnp.clip(jnp.int32(63 + 127) - e, 0, _NUM_VARS - 1)
    parts = []
    for p in range(_PAIRS):
        pidx = (k_all[:, 2 * p] * _NUM_VARS + k_all[:, 2 * p + 1]
                + p * _NUM_VARS * _NUM_VARS)
        parts.append(pidx.reshape(rows // _CHUNK, _CHUNK))
    idx_ref[...] = jnp.concatenate(parts, axis=0)


def _postproject_body(q_ref, wp_ref, bpost_ref, out_ref):
    r = q_ref[...].reshape(_PAIRS, _BLK, _PAIR_DIM)
    acc = None
    for p in range(_PAIRS):
        qp = r[p].astype(jnp.bfloat16)
        wp = wp_ref[p * _PAIR_DIM:(p + 1) * _PAIR_DIM, :].astype(jnp.bfloat16)
        d = jnp.dot(qp, wp, preferred_element_type=jnp.float32)
        acc = d if acc is None else acc + d
    out_ref[...] = acc + bpost_ref[...]


def _postproject_alias_body(q_ref, wp_ref, bpost_ref, prev_ref, out_ref):
    del prev_ref
    _postproject_body(q_ref, wp_ref, bpost_ref, out_ref)


def _make_sc_gather(rows, nch):
    mesh = plsc.VectorSubcoreMesh(core_axis_name="c", subcore_axis_name="s")
    nrows_idx = rows // _CHUNK

    @functools.partial(
        pl.kernel,
        mesh=mesh,
        out_type=jax.ShapeDtypeStruct((rows, _PAIR_DIM), jnp.float32),
        scratch_types=[
            pltpu.VMEM((nrows_idx, _CHUNK), jnp.int32),
            pltpu.VMEM((_CHUNK, _PAIR_DIM), jnp.float32),
            pltpu.VMEM((_CHUNK, _PAIR_DIM), jnp.float32),
            pltpu.SemaphoreType.DMA,
            pltpu.SemaphoreType.DMA,
        ],
    )
    def gather(idx_hbm, pt_hbm, out_hbm, idx_v, buf0, buf1, sem0, sem1):
        wid = lax.axis_index("s") * _NC + lax.axis_index("c")
        pltpu.sync_copy(idx_hbm, idx_v)
        bufs = (buf0, buf1)
        sems = (sem0, sem1)
        base = wid * nch * _CHUNK
        row0 = wid * nch
        cps = [None] * nch
        cps[0] = pltpu.async_copy(pt_hbm.at[idx_v.at[row0]], bufs[0], sems[0])
        for j in range(nch):
            if j + 1 < nch:
                cps[j + 1] = pltpu.async_copy(
                    pt_hbm.at[idx_v.at[row0 + j + 1]],
                    bufs[(j + 1) % 2], sems[(j + 1) % 2])
            cps[j].wait()
            pltpu.sync_copy(bufs[j % 2],
                            out_hbm.at[pl.ds(base + j * _CHUNK, _CHUNK)])

    return gather


def _make_sc_ptbuild():
    mesh = plsc.VectorSubcoreMesh(core_axis_name="c", subcore_axis_name="s")
    L = 16

    @functools.partial(
        pl.kernel,
        mesh=mesh,
        out_type=jax.ShapeDtypeStruct((_PAIRS * _NUM_VARS * _NUM_VARS,
                                       _PAIR_DIM), jnp.float32),
        scratch_types=[
            pltpu.VMEM((8, _VAR_DIM), jnp.float32),
            pltpu.VMEM((_NUM_VARS, _VAR_DIM), jnp.float32),
            pltpu.VMEM((_NUM_VARS, _PAIR_DIM), jnp.float32),
        ],
    )
    def ptbuild(cb_hbm, pt_hbm, cb0_v, cb1_v, buf):
        wid = lax.axis_index("s") * _NC + lax.axis_index("c")
        p = wid // 8
        i0 = (wid % 8) * 8
        pltpu.sync_copy(cb_hbm.at[pl.ds(p * 2 * _NUM_VARS + i0, 8)], cb0_v)
        pltpu.sync_copy(cb_hbm.at[pl.ds(p * 2 * _NUM_VARS + _NUM_VARS,
                                        _NUM_VARS)], cb1_v)

        def fill_cb1(j, _):
            for c in range(_VAR_DIM // L):
                buf[j, pl.ds(_VAR_DIM + c * L, L)] = cb1_v[j, pl.ds(c * L, L)]
            return 0

        lax.fori_loop(0, _NUM_VARS, fill_cb1, 0)
        for i in range(8):
            vs = [cb0_v[i, pl.ds(c * L, L)] for c in range(_VAR_DIM // L)]

            def fill_cb0(j, _, vs=vs):
                for c in range(_VAR_DIM // L):
                    buf[j, pl.ds(c * L, L)] = vs[c]
                return 0

            lax.fori_loop(0, _NUM_VARS, fill_cb0, 0)
            pltpu.sync_copy(
                buf, pt_hbm.at[pl.ds(wid * 512 + i * _NUM_VARS, _NUM_VARS)])

    return ptbuild


def kernel(x, W_pre, b_pre, W_wp, b_wp, codebook, W_post, b_post):
    B, T, IN_DIM = x.shape
    OUT_DIM = W_post.shape[1]
    BT = B * T
    xf = x.reshape(BT, IN_DIM)
    bpre2 = b_pre.reshape(1, -1)
    bwp2 = b_wp.reshape(1, -1)
    bpost2 = b_post.reshape(1, -1)
    NN = _NUM_VARS * _NUM_VARS

    pair_table = _make_sc_ptbuild()(codebook)

    THIRDS = 3
    BT3 = BT // THIRDS
    G3 = BT3 // _BLK
    rows3 = BT3 * _PAIRS
    nch = rows3 // (_NW * _CHUNK)
    sc_gather = _make_sc_gather(rows3, nch)

    pow2 = jnp.asarray(_POW2).astype(jnp.bfloat16)

    def argmax_call(h):
        return pl.pallas_call(
            _logits_argmax_body,
            grid=(G3,),
            in_specs=[
                pl.BlockSpec((_BLK, IN_DIM), lambda i, h=h: (i + G3 * h, 0)),
                pl.BlockSpec((IN_DIM, _PROJ_DIM), lambda i: (0, 0)),
                pl.BlockSpec((1, _PROJ_DIM), lambda i: (0, 0)),
                pl.BlockSpec((_PROJ_DIM, _GROUPS * _NUM_VARS), lambda i: (0, 0)),
                pl.BlockSpec((1, _GROUPS * _NUM_VARS), lambda i: (0, 0)),
                pl.BlockSpec((_GROUPS * _NUM_VARS, _GROUPS), lambda i: (0, 0)),
            ],
            out_specs=pl.BlockSpec((_BLK * _PAIRS // _CHUNK, _CHUNK),
                                   lambda i: (i, 0)),
            out_shape=jax.ShapeDtypeStruct((rows3 // _CHUNK, _CHUNK), jnp.int32),
        )(xf, W_pre, bpre2, W_wp, bwp2, pow2)

    def post_call(h, q, prev):
        qspec = pl.BlockSpec((_BLK * _PAIRS, _PAIR_DIM), lambda i: (i, 0))
        wspec = pl.BlockSpec((_PAIRS * _PAIR_DIM, OUT_DIM), lambda i: (0, 0))
        bspec = pl.BlockSpec((1, OUT_DIM), lambda i: (0, 0))
        ospec = pl.BlockSpec((_BLK, OUT_DIM), lambda i, h=h: (i + G3 * h, 0))
        oshape = jax.ShapeDtypeStruct((BT, OUT_DIM), jnp.float32)
        if prev is None:
            return pl.pallas_call(
                _postproject_body, grid=(G3,),
                in_specs=[qspec, wspec, bspec],
                out_specs=ospec, out_shape=oshape,
            )(q, W_post, bpost2)
        return pl.pallas_call(
            _postproject_alias_body, grid=(G3,),
            in_specs=[qspec, wspec, bspec,
                      pl.BlockSpec(memory_space=pl.ANY)],
            out_specs=ospec, out_shape=oshape,
            input_output_aliases={3: 0},
        )(q, W_post, bpost2, prev)

    idxs = [argmax_call(h) for h in range(THIRDS)]
    qs = [sc_gather(idxs[h], pair_table)
          for h in range(THIRDS)]
    out = None
    for h in range(THIRDS):
        out = post_call(h, qs[h], out)

    return out.reshape(B, T, OUT_DIM)
